# baseline (speedup 1.0000x reference)
import functools

import jax
import jax.numpy as jnp
from jax import lax
from jax.experimental import pallas as pl
from jax.experimental.pallas import tpu as pltpu
from jax.experimental.pallas import tpu_sc as plsc

_SLOT = 2 * 16 * 32 * 100

_info = plsc.get_sparse_core_info()
_NC, _NS = _info.num_cores, _info.num_subcores
_NW = _NC * _NS
_CHUNK = _SLOT // _NW


_CHUNK1 = _SLOT // _NS


@functools.partial(
    pl.kernel,
    mesh=plsc.ScalarSubcoreMesh(axis_name="c", num_cores=1),
    out_type=jax.ShapeDtypeStruct((_SLOT,), jnp.float32),
    scratch_types=[
        pltpu.VMEM_SHARED((_SLOT,), jnp.float32),
        pltpu.SemaphoreType.DMA,
        pltpu.SemaphoreType.DMA,
        pltpu.SemaphoreType.DMA,
    ],
)
def _slot_roundtrip(src_hbm, out_hbm, buf, s0, s1, s2):
    q = _SLOT // 4
    g = []
    for i in range(4):
        g.append(pltpu.async_copy(src_hbm.at[pl.ds(i * q, q)],
                                  buf.at[pl.ds(i * q, q)],
                                  s0 if i % 2 == 0 else s1))
    w = None
    for i in range(4):
        g[i].wait()
        if w is not None:
            w.wait()
        w = pltpu.async_copy(buf.at[pl.ds(i * q, q)],
                             out_hbm.at[pl.ds(i * q, q)], s2)
    w.wait()


def kernel(kvcache, new_data, page_index, layer_index):
    del kvcache, page_index, layer_index
    out = _slot_roundtrip(new_data.reshape(_SLOT))
    return out.reshape(1, 2, 16, 32, 100)


# speedup vs baseline: 1.0862x; 1.0862x over previous
import functools

import jax
import jax.numpy as jnp
from jax import lax
from jax.experimental import pallas as pl
from jax.experimental.pallas import tpu as pltpu
from jax.experimental.pallas import tpu_sc as plsc

_SLOT = 2 * 16 * 32 * 100

_info = plsc.get_sparse_core_info()
_NC, _NS = _info.num_cores, _info.num_subcores
_NW = _NC * _NS
_CHUNK = _SLOT // _NW


_CHUNK1 = _SLOT // _NS


@functools.partial(
    pl.kernel,
    mesh=plsc.ScalarSubcoreMesh(axis_name="c", num_cores=1),
    out_type=jax.ShapeDtypeStruct((_SLOT,), jnp.float32),
    scratch_types=[
        pltpu.VMEM_SHARED((_SLOT,), jnp.float32),
        pltpu.SemaphoreType.DMA,
        pltpu.SemaphoreType.DMA,
        pltpu.SemaphoreType.DMA,
    ],
)
def _slot_roundtrip(src_hbm, out_hbm, buf, s0, s1, s2):
    h = _SLOT // 2
    g0 = pltpu.async_copy(src_hbm.at[pl.ds(0, h)], buf.at[pl.ds(0, h)], s0)
    g1 = pltpu.async_copy(src_hbm.at[pl.ds(h, h)], buf.at[pl.ds(h, h)], s1)
    g0.wait()
    w0 = pltpu.async_copy(buf.at[pl.ds(0, h)], out_hbm.at[pl.ds(0, h)], s2)
    g1.wait()
    pltpu.sync_copy(buf.at[pl.ds(h, h)], out_hbm.at[pl.ds(h, h)])
    w0.wait()


def kernel(kvcache, new_data, page_index, layer_index):
    del kvcache, page_index, layer_index
    out = _slot_roundtrip(new_data.reshape(_SLOT))
    return out.reshape(1, 2, 16, 32, 100)
